# Initial kernel scaffold; baseline (speedup 1.0000x reference)
import jax
import jax.numpy as jnp
from jax.experimental import pallas as pl


def kernel(indices, table, gamma, beta):
    raise NotImplementedError("write your pallas kernel here")


# speedup vs baseline: 4.2360x; 4.2360x over previous
import functools

import jax
import jax.numpy as jnp
from jax import lax
from jax.experimental import pallas as pl
from jax.experimental.pallas import tpu as pltpu
from jax.experimental.pallas import tpu_sc as plsc

_NC = 2
_NS = 16
_NW = _NC * _NS

_ROWS = 128
_K = 8


def _ln_body(tab_ref, g_ref, b_ref, out_ref):
    x = tab_ref[...]
    mean = jnp.mean(x, axis=1, keepdims=True)
    c = x - mean
    var = jnp.mean(c * c, axis=1, keepdims=True)
    out_ref[...] = c * lax.rsqrt(var + 1e-5) * g_ref[...] + b_ref[...]


def _normalize_table(table, gamma, beta):
    v, d = table.shape
    blk = 8000
    return pl.pallas_call(
        _ln_body,
        grid=(v // blk,),
        in_specs=[
            pl.BlockSpec((blk, d), lambda i: (i, 0)),
            pl.BlockSpec((1, d), lambda i: (0, 0)),
            pl.BlockSpec((1, d), lambda i: (0, 0)),
        ],
        out_specs=pl.BlockSpec((blk, d), lambda i: (i, 0)),
        out_shape=jax.ShapeDtypeStruct((v, d), jnp.float32),
    )(table, gamma.reshape(1, d), beta.reshape(1, d))


def _sc_gather(tab, idx2d):
    n_chunks, rows = idx2d.shape
    d = tab.shape[1]
    cpw = n_chunks // _NW
    n_groups = cpw // _K
    n_pairs = n_groups // 2

    mesh = plsc.VectorSubcoreMesh(core_axis_name="c", subcore_axis_name="s")

    @functools.partial(
        pl.kernel,
        mesh=mesh,
        out_type=jax.ShapeDtypeStruct((n_chunks, rows, d), jnp.float32),
        compiler_params=pltpu.CompilerParams(use_tc_tiling_on_sc=False),
        scratch_types=[
            pltpu.VMEM((2, _K, rows), jnp.int32),
            pltpu.VMEM((2, _K, rows, d), jnp.float32),
            pltpu.SemaphoreType.DMA,
            pltpu.SemaphoreType.DMA,
        ],
    )
    def k(tab_hbm, idx_hbm, out_hbm, idx_v, rows_v, sem0, sem1):
        wid = lax.axis_index("s") * _NC + lax.axis_index("c")
        base = wid * cpw
        sems = (sem0, sem1)

        def load_idx(b, g):
            pltpu.sync_copy(idx_hbm.at[pl.ds(base + g * _K, _K)], idx_v.at[b])

        def fire(b):
            for j in range(_K):
                pltpu.make_async_copy(
                    tab_hbm.at[idx_v.at[b, j]], rows_v.at[b, j], sems[b]
                ).start()

        def drain(b):
            for j in range(_K):
                pltpu.make_async_copy(
                    tab_hbm.at[idx_v.at[b, j]], rows_v.at[b, j], sems[b]
                ).wait()

        def store(b, g):
            pltpu.sync_copy(rows_v.at[b], out_hbm.at[pl.ds(base + g * _K, _K)])

        load_idx(0, 0)
        fire(0)

        def pair(i, carry):
            g_a = 2 * i
            g_b = g_a + 1
            load_idx(1, g_b)
            fire(1)
            drain(0)
            store(0, g_a)

            @pl.when(i + 1 < n_pairs)
            def _():
                load_idx(0, g_a + 2)
                fire(0)

            drain(1)
            store(1, g_b)
            return carry

        lax.fori_loop(0, n_pairs, pair, None)

    return k(tab, idx2d)


def kernel(indices, table, gamma, beta):
    d = table.shape[1]
    norm = _normalize_table(table, gamma, beta)
    flat = indices.reshape(-1).astype(jnp.int32)
    idx2d = flat.reshape(flat.shape[0] // _ROWS, _ROWS)
    out = _sc_gather(norm, idx2d)
    return out.reshape(indices.shape + (d,))


# speedup vs baseline: 4.5717x; 1.0792x over previous
import functools

import jax
import jax.numpy as jnp
from jax import lax
from jax.experimental import pallas as pl
from jax.experimental.pallas import tpu as pltpu
from jax.experimental.pallas import tpu_sc as plsc

_NC = 2
_NS = 16
_NW = _NC * _NS

_GROW = 100
_RPG = 4


def _ln_t_body(tab_ref, g_ref, b_ref, out_ref):
    x = tab_ref[...]
    mean = jnp.mean(x, axis=0, keepdims=True)
    c = x - mean
    var = jnp.mean(c * c, axis=0, keepdims=True)
    out_ref[...] = c * lax.rsqrt(var + 1e-5) * g_ref[...] + b_ref[...]


def _normalize_table_t(table_t, gamma, beta):
    d, v = table_t.shape
    blk = 16384
    return pl.pallas_call(
        _ln_t_body,
        grid=(pl.cdiv(v, blk),),
        in_specs=[
            pl.BlockSpec((d, blk), lambda i: (0, i)),
            pl.BlockSpec((d, 1), lambda i: (0, 0)),
            pl.BlockSpec((d, 1), lambda i: (0, 0)),
        ],
        out_specs=pl.BlockSpec((d, blk), lambda i: (0, i)),
        out_shape=jax.ShapeDtypeStruct((d, v), jnp.float32),
    )(table_t, gamma.reshape(d, 1), beta.reshape(d, 1))


def _sc_gather(tab, idx2d, batch, hist):
    d = tab.shape[1]
    rows_per_w = batch // _NW
    n_groups = rows_per_w // _RPG
    n_pairs = n_groups // 2
    dpr = hist // _GROW
    dpg = _RPG * dpr
    ipg = _RPG * hist // _GROW

    mesh = plsc.VectorSubcoreMesh(core_axis_name="c", subcore_axis_name="s")

    @functools.partial(
        pl.kernel,
        mesh=mesh,
        out_type=jax.ShapeDtypeStruct((batch, hist, d), jnp.float32),
        compiler_params=pltpu.CompilerParams(use_tc_tiling_on_sc=False),
        scratch_types=[
            pltpu.VMEM((2, dpg, _GROW), jnp.int32),
            pltpu.VMEM((2, _RPG, hist, d), jnp.float32),
            pltpu.SemaphoreType.DMA,
            pltpu.SemaphoreType.DMA,
        ],
    )
    def k(tab_hbm, idx_hbm, out_hbm, idx_v, rows_v, sem0, sem1):
        wid = lax.axis_index("s") * _NC + lax.axis_index("c")
        ibase = wid * n_groups * ipg
        obase = wid * rows_per_w
        sems = (sem0, sem1)

        def load_idx(b, g):
            pltpu.sync_copy(idx_hbm.at[pl.ds(ibase + g * ipg, ipg)], idx_v.at[b])

        def descs(b):
            for j in range(dpg):
                yield (
                    tab_hbm.at[idx_v.at[b, j]],
                    rows_v.at[b, j // dpr, pl.ds((j % dpr) * _GROW, _GROW)],
                    sems[b],
                )

        def fire(b):
            for src, dst, sem in descs(b):
                pltpu.make_async_copy(src, dst, sem).start()

        def drain(b):
            for src, dst, sem in descs(b):
                pltpu.make_async_copy(src, dst, sem).wait()

        def store(b, g):
            pltpu.sync_copy(rows_v.at[b], out_hbm.at[pl.ds(obase + g * _RPG, _RPG)])

        load_idx(0, 0)
        fire(0)

        def pair(i, carry):
            g_a = 2 * i
            g_b = g_a + 1
            load_idx(1, g_b)
            fire(1)
            drain(0)
            store(0, g_a)

            @pl.when(i + 1 < n_pairs)
            def _():
                load_idx(0, g_a + 2)
                fire(0)

            drain(1)
            store(1, g_b)
            return carry

        lax.fori_loop(0, n_pairs, pair, None)

    return k(tab, idx2d)


def kernel(indices, table, gamma, beta):
    batch, hist = indices.shape
    d = table.shape[1]
    norm_t = _normalize_table_t(table.T, gamma, beta)
    idx2d = indices.astype(jnp.int32).reshape(batch * hist // _GROW, _GROW)
    return _sc_gather(norm_t.T, idx2d, batch, hist)


# speedup vs baseline: 7.3853x; 1.6155x over previous
import functools

import jax
import jax.numpy as jnp
from jax import lax
from jax.experimental import pallas as pl
from jax.experimental.pallas import tpu as pltpu
from jax.experimental.pallas import tpu_sc as plsc

_NC = 2
_NS = 16
_NW = _NC * _NS

_GROW = 100
_RPG = 4


def _ln_t_body(tab_ref, g_ref, b_ref, out_ref):
    x = tab_ref[...]
    mean = jnp.mean(x, axis=0, keepdims=True)
    c = x - mean
    var = jnp.mean(c * c, axis=0, keepdims=True)
    out_ref[...] = c * lax.rsqrt(var + 1e-5) * g_ref[...] + b_ref[...]


def _normalize_table_t(table_t, gamma, beta):
    d, v = table_t.shape
    blk = 16384
    return pl.pallas_call(
        _ln_t_body,
        grid=(pl.cdiv(v, blk),),
        in_specs=[
            pl.BlockSpec((d, blk), lambda i: (0, i)),
            pl.BlockSpec((d, 1), lambda i: (0, 0)),
            pl.BlockSpec((d, 1), lambda i: (0, 0)),
        ],
        out_specs=pl.BlockSpec((d, blk), lambda i: (0, i)),
        out_shape=jax.ShapeDtypeStruct((d, v), jnp.float32),
    )(table_t, gamma.reshape(d, 1), beta.reshape(d, 1))


def _sc_gather(tab, idx2d, batch, hist):
    d = tab.shape[1]
    rows_per_w = batch // _NW
    n_groups = rows_per_w // _RPG
    n_pairs = n_groups // 2
    dpr = hist // _GROW
    dpg = _RPG * dpr
    ipg = _RPG * hist // _GROW

    mesh = plsc.VectorSubcoreMesh(core_axis_name="c", subcore_axis_name="s")

    @functools.partial(
        pl.kernel,
        mesh=mesh,
        out_type=jax.ShapeDtypeStruct((batch, hist, 128), jnp.float32),
        compiler_params=pltpu.CompilerParams(use_tc_tiling_on_sc=False),
        scratch_types=[
            pltpu.VMEM((2, dpg, _GROW), jnp.int32),
            pltpu.VMEM((2, _RPG, hist, d), jnp.float32),
            pltpu.SemaphoreType.DMA,
            pltpu.SemaphoreType.DMA,
        ],
    )
    def k(tab_hbm, idx_hbm, out_hbm, idx_v, rows_v, sem0, sem1):
        wid = lax.axis_index("s") * _NC + lax.axis_index("c")
        ibase = wid * n_groups * ipg
        obase = wid * rows_per_w
        sems = (sem0, sem1)

        def load_idx(b, g):
            pltpu.sync_copy(idx_hbm.at[pl.ds(ibase + g * ipg, ipg)], idx_v.at[b])

        def descs(b):
            for j in range(dpg):
                yield (
                    tab_hbm.at[idx_v.at[b, j]],
                    rows_v.at[b, j // dpr, pl.ds((j % dpr) * _GROW, _GROW)],
                    sems[b],
                )

        def fire(b):
            for src, dst, sem in descs(b):
                pltpu.make_async_copy(src, dst, sem).start()

        def drain(b):
            for src, dst, sem in descs(b):
                pltpu.make_async_copy(src, dst, sem).wait()

        def store(b, g):
            pltpu.sync_copy(
                rows_v.at[b],
                out_hbm.at[pl.ds(obase + g * _RPG, _RPG), :, pl.ds(0, d)],
            )

        load_idx(0, 0)
        fire(0)

        def pair(i, carry):
            g_a = 2 * i
            g_b = g_a + 1
            load_idx(1, g_b)
            fire(1)
            drain(0)
            store(0, g_a)

            @pl.when(i + 1 < n_pairs)
            def _():
                load_idx(0, g_a + 2)
                fire(0)

            drain(1)
            store(1, g_b)
            return carry

        lax.fori_loop(0, n_pairs, pair, None)

    return k(tab, idx2d)


def kernel(indices, table, gamma, beta):
    batch, hist = indices.shape
    d = table.shape[1]
    norm_t = _normalize_table_t(table.T, gamma, beta)
    idx2d = indices.astype(jnp.int32).reshape(batch * hist // _GROW, _GROW)
    padded = _sc_gather(norm_t.T, idx2d, batch, hist)
    return padded[:, :, :d]
